# baseline (speedup 1.0000x reference)
import dataclasses
import functools

import jax
import jax.numpy as jnp
import numpy as np
from jax import lax
from jax.experimental import pallas as pl
from jax.experimental.pallas import tpu as pltpu
from jax.experimental.pallas import tpu_sc as plsc

_EMBED = 256
_NH = 8
_HD = 32
_PTS = 12
_LVLS = ((80, 80), (40, 40), (20, 20))
_BS = 16
_LQ = 1000
_K = _NH * _PTS
_ROWS = _BS * _LQ
_SEG = sum(h * w for h, w in _LVLS)
_TBL = _BS * _NH * _SEG
_G = 4 * _K

_NW = 32
_QPW = _ROWS // _NW
_QB = 2
_NB = _QPW // _QB
_GB = _QB * _G


def _prep_body(q_ref, rp_ref, wx_ref, wy_ref, wa_ref, bx_ref, by_ref, ba_ref,
               seg_ref, wf_ref, hf_ref, lb2_ref, bmul_ref, wi_ref,
               idx_ref, wts_ref):
    b = pl.program_id(0)
    q = q_ref[...]
    offx = jnp.dot(q, wx_ref[...], preferred_element_type=jnp.float32) + bx_ref[...]
    offy = jnp.dot(q, wy_ref[...], preferred_element_type=jnp.float32) + by_ref[...]
    logit = jnp.dot(q, wa_ref[...], preferred_element_type=jnp.float32) + ba_ref[...]

    m = jnp.max(logit, axis=1, keepdims=True)
    e = jnp.exp(logit - m)
    ssum = jnp.dot(e, seg_ref[...], preferred_element_type=jnp.float32)
    attn = e / ssum

    cx = rp_ref[:, 0:1]
    cy = rp_ref[:, 1:2]
    rw = rp_ref[:, 2:3]
    rh = rp_ref[:, 3:4]
    wf = wf_ref[...]
    hf = hf_ref[...]
    x = (cx + offx * (0.125 * rw)) * wf - 0.5
    y = (cy + offy * (0.125 * rh)) * hf - 0.5
    x0 = jnp.floor(x)
    y0 = jnp.floor(y)
    fx1 = x - x0
    fx0 = 1.0 - fx1
    fy1 = y - y0
    fy0 = 1.0 - fy1

    base = b * bmul_ref[...] + lb2_ref[...]
    wi = wi_ref[...]

    idx_parts = []
    wts_parts = []
    for dx, dy, fx, fy in ((0, 0, fx0, fy0), (1, 0, fx1, fy0),
                           (0, 1, fx0, fy1), (1, 1, fx1, fy1)):
        xi = x0 + dx
        yi = y0 + dy
        valid = ((xi >= 0.0) & (xi <= wf - 1.0)
                 & (yi >= 0.0) & (yi <= hf - 1.0))
        xc = jnp.clip(xi, 0.0, wf - 1.0).astype(jnp.int32)
        yc = jnp.clip(yi, 0.0, hf - 1.0).astype(jnp.int32)
        ind = base + yc * wi + xc
        idx_parts.append(jnp.clip(ind, 0, _TBL - 1))
        wts_parts.append(jnp.where(valid, fx * fy, 0.0) * attn)
    idx_ref[...] = jnp.concatenate(idx_parts, axis=1)
    wts_ref[...] = jnp.concatenate(wts_parts, axis=1)


def _prep(query2, rp2, wx, wy, wa, bx, by, ba, seg, wf, hf, lb2, bmul, wi):
    rep = lambda b: (0, 0)
    return pl.pallas_call(
        _prep_body,
        grid=(_BS,),
        in_specs=[
            pl.BlockSpec((_LQ, _EMBED), lambda b: (b, 0)),
            pl.BlockSpec((_LQ, 4), lambda b: (b, 0)),
            pl.BlockSpec((_EMBED, _K), rep),
            pl.BlockSpec((_EMBED, _K), rep),
            pl.BlockSpec((_EMBED, _K), rep),
            pl.BlockSpec((1, _K), rep),
            pl.BlockSpec((1, _K), rep),
            pl.BlockSpec((1, _K), rep),
            pl.BlockSpec((_K, _K), rep),
            pl.BlockSpec((1, _K), rep),
            pl.BlockSpec((1, _K), rep),
            pl.BlockSpec((1, _K), rep),
            pl.BlockSpec((1, _K), rep),
            pl.BlockSpec((1, _K), rep),
        ],
        out_specs=[
            pl.BlockSpec((_LQ, _G), lambda b: (b, 0)),
            pl.BlockSpec((_LQ, _G), lambda b: (b, 0)),
        ],
        out_shape=[
            jax.ShapeDtypeStruct((_ROWS, _G), jnp.int32),
            jax.ShapeDtypeStruct((_ROWS, _G), jnp.float32),
        ],
    )(query2, rp2, wx, wy, wa, bx, by, ba, seg, wf, hf, lb2, bmul, wi)


def _sc_body(tbl_hbm, idx_hbm, wts_hbm, out_hbm,
             idx_v0, idx_v1, idx_v2, wts_v0, wts_v1, wts_v2,
             rows_v0, rows_v1, rows_v2, out_v0, out_v1, out_v2,
             sl0, sl1, sl2, sg0, sg1, sg2, ss0, ss1, ss2):
    c = lax.axis_index("c")
    s = lax.axis_index("s")
    wid = c * 16 + s
    row0 = wid * _QPW
    bufs = ((idx_v0, wts_v0, rows_v0, out_v0, sl0, sg0, ss0),
            (idx_v1, wts_v1, rows_v1, out_v1, sl1, sg1, ss1),
            (idx_v2, wts_v2, rows_v2, out_v2, sl2, sg2, ss2))

    def load(blk, buf):
        idx_v, wts_v, _, _, sl, _, _ = buf
        off = (row0 + blk * _QB) * _G
        pltpu.async_copy(idx_hbm.at[pl.ds(off, _GB)], idx_v, sl)
        pltpu.async_copy(wts_hbm.at[pl.ds(off, _GB)], wts_v, sl)

    def wait_load(buf):
        idx_v, wts_v, _, _, sl, _, _ = buf
        pltpu.make_async_copy(idx_hbm.at[pl.ds(0, _GB)], idx_v, sl).wait()
        pltpu.make_async_copy(wts_hbm.at[pl.ds(0, _GB)], wts_v, sl).wait()

    def gather(buf):
        idx_v, _, rows_v, _, _, sg, _ = buf
        pltpu.async_copy(tbl_hbm.at[idx_v], rows_v, sg)

    def wait_gather(buf):
        idx_v, _, rows_v, _, _, sg, _ = buf
        pltpu.make_async_copy(tbl_hbm.at[idx_v], rows_v, sg).wait()

    def store(blk, buf):
        _, _, _, out_v, _, _, ss = buf
        qrow = row0 + blk * _QB
        pltpu.async_copy(out_v, out_hbm.at[pl.ds(qrow * _NH, _QB * _NH)], ss)

    def wait_store(buf):
        _, _, _, out_v, _, _, ss = buf
        pltpu.make_async_copy(
            out_v, out_hbm.at[pl.ds(0, _QB * _NH)], ss).wait()

    def compute(blk, buf):
        _, wts_v, rows_v, out_v, _, _, _ = buf
        lanes = lax.iota(jnp.int32, 16)
        for qi in range(_QB):
            @pl.loop(0, _NH)
            def _(h):
                jbase = qi * _G + h * _PTS

                def body(p, accs):
                    a0, a1 = accs
                    for cc in range(4):
                        j = jbase + cc * _K + p
                        wv = plsc.load_gather(
                            wts_v, [jnp.broadcast_to(j, (16,))])
                        ra, rb = plsc.unpack(rows_v[j],
                                             format=plsc.PackFormat.INTERLEAVED)
                        a0 = a0 + wv * ra
                        a1 = a1 + wv * rb
                    return (a0, a1)

                a0, a1 = lax.fori_loop(
                    0, _PTS, body,
                    (jnp.zeros((16,), jnp.float32),
                     jnp.zeros((16,), jnp.float32)))
                orow = out_v.at[qi * _NH + h]
                plsc.store_scatter(orow, [lanes * 2], a0)
                plsc.store_scatter(orow, [lanes * 2 + 1], a1)
        store(blk, buf)

    load(0, bufs[0])
    load(1, bufs[1])
    wait_load(bufs[0])
    gather(bufs[0])

    ngrp = (_NB - 1) // 3

    @pl.loop(0, ngrp)
    def _(i):
        for r in range(3):
            b = 3 * i + r
            buf = bufs[r]
            wait_gather(buf)
            nbuf = bufs[(r + 1) % 3]
            wait_load(nbuf)
            gather(nbuf)

            @pl.when(b + 2 < _NB)
            def _():
                load(b + 2, bufs[(r + 2) % 3])

            @pl.when(b >= 3)
            def _():
                wait_store(buf)

            compute(b, buf)
            store(b, buf)

    wait_gather(bufs[0])
    wait_store(bufs[0])
    compute(_NB - 1, bufs[0])
    store(_NB - 1, bufs[0])

    wait_store(bufs[0])
    wait_store(bufs[1])
    wait_store(bufs[2])


def _sc_gather(tbl, idx3, wts_flat):
    mesh = plsc.VectorSubcoreMesh(core_axis_name="c", subcore_axis_name="s")
    cp = pltpu.CompilerParams()
    if "needs_layout_passes" in pltpu.CompilerParams.__dataclass_fields__:
        cp = dataclasses.replace(cp, needs_layout_passes=False)
    if "use_tc_tiling_on_sc" in pltpu.CompilerParams.__dataclass_fields__:
        cp = dataclasses.replace(cp, use_tc_tiling_on_sc=False)
    f = pl.kernel(
        _sc_body,
        mesh=mesh,
        compiler_params=cp,
        out_type=jax.ShapeDtypeStruct((_ROWS * _NH, _HD), jnp.float32),
        scratch_types=(
            [pltpu.VMEM((_GB,), jnp.int32)] * 3
            + [pltpu.VMEM((_GB,), jnp.float32)] * 3
            + [pltpu.VMEM((_GB, _HD), jnp.bfloat16)] * 3
            + [pltpu.VMEM((_QB * _NH, _HD), jnp.float32)] * 3
            + [pltpu.SemaphoreType.DMA] * 9
        ),
    )
    return f(tbl, idx3, wts_flat)


def kernel(query, ref_pts, value_0, value_1, value_2, W_off, b_off, W_attn,
           b_attn):
    wx = W_off[:, 0::2]
    wy = W_off[:, 1::2]
    bx = b_off[0::2][None, :]
    by = b_off[1::2][None, :]
    ba = b_attn[None, :]

    seg = jnp.asarray(np.kron(np.eye(_NH, dtype=np.float32),
                              np.ones((_PTS, _PTS), np.float32)))
    lvl_of_p = np.repeat(np.arange(3), 4)
    hw_np = np.array([h * w for h, w in _LVLS], np.int64)
    wnp = np.array([w for _, w in _LVLS], np.float32)[lvl_of_p]
    hnp = np.array([h for h, _ in _LVLS], np.float32)[lvl_of_p]
    lvl_base = np.concatenate(([0], np.cumsum(hw_np * _BS * _NH)[:-1]))
    kk = np.arange(_K)
    hh = kk // _PTS
    ll = lvl_of_p[kk % _PTS]
    wf = jnp.asarray(np.tile(wnp, _NH)[None, :])
    hf = jnp.asarray(np.tile(hnp, _NH)[None, :])
    wi = jnp.asarray(np.tile(wnp, _NH)[None, :].astype(np.int32))
    lb2 = jnp.asarray((lvl_base[ll] + hh * hw_np[ll]).astype(np.int32)[None, :])
    bmul = jnp.asarray((_NH * hw_np[ll]).astype(np.int32)[None, :])

    query2 = query.reshape(_ROWS, _EMBED)
    rp2 = ref_pts.reshape(_ROWS, 4)
    idx, wts = _prep(query2, rp2, wx, wy, W_attn, bx, by, ba, seg, wf, hf,
                     lb2, bmul, wi)

    tbl = jnp.concatenate(
        [v.astype(jnp.bfloat16).transpose(0, 2, 3, 1).reshape(-1, _HD)
         for v in (value_0, value_1, value_2)], axis=0)

    idx_flat = idx.reshape(-1)
    wts_flat = wts.reshape(-1)
    out = _sc_gather(tbl, idx_flat, wts_flat)
    return out.reshape(_BS, _LQ, _NH * _HD)


# speedup vs baseline: 1.2806x; 1.2806x over previous
import dataclasses
import functools

import jax
import jax.numpy as jnp
import numpy as np
from jax import lax
from jax.experimental import pallas as pl
from jax.experimental.pallas import tpu as pltpu
from jax.experimental.pallas import tpu_sc as plsc

_EMBED = 256
_NH = 8
_HD = 32
_PTS = 12
_LVLS = ((80, 80), (40, 40), (20, 20))
_BS = 16
_LQ = 1000
_K = _NH * _PTS
_ROWS = _BS * _LQ
_SEG = sum(h * w for h, w in _LVLS)
_TBL = _BS * _NH * _SEG
_G = 4 * _K

_NW = 32
_QPW = _ROWS // _NW
_QB = 2
_NB = _QPW // _QB
_GB = _QB * _G


def _prep_body(q_ref, rp_ref, wx_ref, wy_ref, wa_ref, bx_ref, by_ref, ba_ref,
               seg_ref, wf_ref, hf_ref, lb2_ref, bmul_ref, wi_ref,
               idx_ref, wts_ref):
    b = pl.program_id(0)
    q = q_ref[...]
    offx = jnp.dot(q, wx_ref[...], preferred_element_type=jnp.float32) + bx_ref[...]
    offy = jnp.dot(q, wy_ref[...], preferred_element_type=jnp.float32) + by_ref[...]
    logit = jnp.dot(q, wa_ref[...], preferred_element_type=jnp.float32) + ba_ref[...]

    m = jnp.max(logit, axis=1, keepdims=True)
    e = jnp.exp(logit - m)
    ssum = jnp.dot(e, seg_ref[...], preferred_element_type=jnp.float32)
    attn = e / ssum

    cx = rp_ref[:, 0:1]
    cy = rp_ref[:, 1:2]
    rw = rp_ref[:, 2:3]
    rh = rp_ref[:, 3:4]
    wf = wf_ref[...]
    hf = hf_ref[...]
    x = (cx + offx * (0.125 * rw)) * wf - 0.5
    y = (cy + offy * (0.125 * rh)) * hf - 0.5
    x0 = jnp.floor(x)
    y0 = jnp.floor(y)
    fx1 = x - x0
    fx0 = 1.0 - fx1
    fy1 = y - y0
    fy0 = 1.0 - fy1

    base = b * bmul_ref[...] + lb2_ref[...]
    wi = wi_ref[...]

    idx_parts = []
    wts_parts = []
    for dx, dy, fx, fy in ((0, 0, fx0, fy0), (1, 0, fx1, fy0),
                           (0, 1, fx0, fy1), (1, 1, fx1, fy1)):
        xi = x0 + dx
        yi = y0 + dy
        valid = ((xi >= 0.0) & (xi <= wf - 1.0)
                 & (yi >= 0.0) & (yi <= hf - 1.0))
        xc = jnp.clip(xi, 0.0, wf - 1.0).astype(jnp.int32)
        yc = jnp.clip(yi, 0.0, hf - 1.0).astype(jnp.int32)
        ind = base + yc * wi + xc
        idx_parts.append(jnp.clip(ind, 0, _TBL - 1))
        wts_parts.append(jnp.where(valid, fx * fy, 0.0) * attn)
    idx_ref[...] = jnp.concatenate(idx_parts, axis=1)
    wts_ref[...] = jnp.concatenate(wts_parts, axis=1)


def _prep(query2, rp2, wx, wy, wa, bx, by, ba, seg, wf, hf, lb2, bmul, wi):
    rep = lambda b: (0, 0)
    return pl.pallas_call(
        _prep_body,
        grid=(_BS,),
        in_specs=[
            pl.BlockSpec((_LQ, _EMBED), lambda b: (b, 0)),
            pl.BlockSpec((_LQ, 4), lambda b: (b, 0)),
            pl.BlockSpec((_EMBED, _K), rep),
            pl.BlockSpec((_EMBED, _K), rep),
            pl.BlockSpec((_EMBED, _K), rep),
            pl.BlockSpec((1, _K), rep),
            pl.BlockSpec((1, _K), rep),
            pl.BlockSpec((1, _K), rep),
            pl.BlockSpec((_K, _K), rep),
            pl.BlockSpec((1, _K), rep),
            pl.BlockSpec((1, _K), rep),
            pl.BlockSpec((1, _K), rep),
            pl.BlockSpec((1, _K), rep),
            pl.BlockSpec((1, _K), rep),
        ],
        out_specs=[
            pl.BlockSpec((_LQ, _G), lambda b: (b, 0)),
            pl.BlockSpec((_LQ, _G), lambda b: (b, 0)),
        ],
        out_shape=[
            jax.ShapeDtypeStruct((_ROWS, _G), jnp.int32),
            jax.ShapeDtypeStruct((_ROWS, _G), jnp.float32),
        ],
    )(query2, rp2, wx, wy, wa, bx, by, ba, seg, wf, hf, lb2, bmul, wi)


def _sc_body(tbl_hbm, idx_hbm, wts_hbm, out_hbm,
             idx_v0, idx_v1, idx_v2, wts_v0, wts_v1, wts_v2,
             rows_v0, rows_v1, rows_v2, out_v0, out_v1, out_v2,
             sl0, sl1, sl2, sg0, sg1, sg2, ss0, ss1, ss2):
    c = lax.axis_index("c")
    s = lax.axis_index("s")
    wid = c * 16 + s
    row0 = wid * _QPW
    bufs = ((idx_v0, wts_v0, rows_v0, out_v0, sl0, sg0, ss0),
            (idx_v1, wts_v1, rows_v1, out_v1, sl1, sg1, ss1),
            (idx_v2, wts_v2, rows_v2, out_v2, sl2, sg2, ss2))

    def load(blk, buf):
        idx_v, wts_v, _, _, sl, _, _ = buf
        off = (row0 + blk * _QB) * _G
        pltpu.async_copy(idx_hbm.at[pl.ds(off, _GB)], idx_v, sl)
        pltpu.async_copy(wts_hbm.at[pl.ds(off, _GB)], wts_v, sl)

    def wait_load(buf):
        idx_v, wts_v, _, _, sl, _, _ = buf
        pltpu.make_async_copy(idx_hbm.at[pl.ds(0, _GB)], idx_v, sl).wait()
        pltpu.make_async_copy(wts_hbm.at[pl.ds(0, _GB)], wts_v, sl).wait()

    def gather(buf):
        idx_v, _, rows_v, _, _, sg, _ = buf
        pltpu.async_copy(tbl_hbm.at[idx_v], rows_v, sg)

    def wait_gather(buf):
        idx_v, _, rows_v, _, _, sg, _ = buf
        pltpu.make_async_copy(tbl_hbm.at[idx_v], rows_v, sg).wait()

    def store(blk, buf):
        _, _, _, out_v, _, _, ss = buf
        qrow = row0 + blk * _QB
        pltpu.async_copy(out_v, out_hbm.at[pl.ds(qrow * _NH, _QB * _NH)], ss)

    def wait_store(buf):
        _, _, _, out_v, _, _, ss = buf
        pltpu.make_async_copy(
            out_v, out_hbm.at[pl.ds(0, _QB * _NH)], ss).wait()

    def compute(blk, buf):
        _, wts_v, rows_v, out_v, _, _, _ = buf
        lanes = lax.iota(jnp.int32, 16)
        for qi in range(_QB):
            @pl.loop(0, _NH)
            def _(h):
                jbase = qi * _G + h * _PTS

                def body(p, accs):
                    a0, a1 = accs
                    for cc in range(4):
                        j = jbase + cc * _K + p
                        wv = plsc.load_gather(
                            wts_v, [jnp.broadcast_to(j, (16,))])
                        ra, rb = plsc.unpack(rows_v[j],
                                             format=plsc.PackFormat.INTERLEAVED)
                        a0 = a0 + wv * ra
                        a1 = a1 + wv * rb
                    return (a0, a1)

                a0, a1 = lax.fori_loop(
                    0, _PTS, body,
                    (jnp.zeros((16,), jnp.float32),
                     jnp.zeros((16,), jnp.float32)))
                orow = out_v.at[qi * _NH + h]
                plsc.store_scatter(orow, [lanes * 2], a0)
                plsc.store_scatter(orow, [lanes * 2 + 1], a1)
        store(blk, buf)

    load(0, bufs[0])
    load(1, bufs[1])
    wait_load(bufs[0])
    gather(bufs[0])

    ngrp = (_NB - 1) // 3

    @pl.loop(0, ngrp)
    def _(i):
        for r in range(3):
            b = 3 * i + r
            buf = bufs[r]
            wait_gather(buf)
            nbuf = bufs[(r + 1) % 3]
            wait_load(nbuf)
            gather(nbuf)

            @pl.when(b + 2 < _NB)
            def _():
                load(b + 2, bufs[(r + 2) % 3])

            @pl.when(b >= 3)
            def _():
                wait_store(buf)

            compute(b, buf)
            store(b, buf)

    wait_gather(bufs[0])
    wait_store(bufs[0])
    compute(_NB - 1, bufs[0])
    store(_NB - 1, bufs[0])

    wait_store(bufs[0])
    wait_store(bufs[1])
    wait_store(bufs[2])


def _sc_gather(tbl, idx3, wts_flat):
    mesh = plsc.VectorSubcoreMesh(core_axis_name="c", subcore_axis_name="s")
    cp = pltpu.CompilerParams()
    if "needs_layout_passes" in pltpu.CompilerParams.__dataclass_fields__:
        cp = dataclasses.replace(cp, needs_layout_passes=False)
    if "use_tc_tiling_on_sc" in pltpu.CompilerParams.__dataclass_fields__:
        cp = dataclasses.replace(cp, use_tc_tiling_on_sc=False)
    f = pl.kernel(
        _sc_body,
        mesh=mesh,
        compiler_params=cp,
        out_type=jax.ShapeDtypeStruct((_ROWS * _NH, _HD), jnp.float32),
        scratch_types=(
            [pltpu.VMEM((_GB,), jnp.int32)] * 3
            + [pltpu.VMEM((_GB,), jnp.float32)] * 3
            + [pltpu.VMEM((_GB, _HD), jnp.bfloat16)] * 3
            + [pltpu.VMEM((_QB * _NH, _HD), jnp.float32)] * 3
            + [pltpu.SemaphoreType.DMA] * 9
        ),
    )
    return f(tbl, idx3, wts_flat)


def kernel(query, ref_pts, value_0, value_1, value_2, W_off, b_off, W_attn,
           b_attn):
    wx = W_off[:, 0::2]
    wy = W_off[:, 1::2]
    bx = b_off[0::2][None, :]
    by = b_off[1::2][None, :]
    ba = b_attn[None, :]

    seg = jnp.asarray(np.kron(np.eye(_NH, dtype=np.float32),
                              np.ones((_PTS, _PTS), np.float32)))
    lvl_of_p = np.repeat(np.arange(3), 4)
    hw_np = np.array([h * w for h, w in _LVLS], np.int64)
    wnp = np.array([w for _, w in _LVLS], np.float32)[lvl_of_p]
    hnp = np.array([h for h, _ in _LVLS], np.float32)[lvl_of_p]
    lvl_off = np.concatenate(([0], np.cumsum(hw_np)[:-1]))
    kk = np.arange(_K)
    hh = kk // _PTS
    ll = lvl_of_p[kk % _PTS]
    wf = jnp.asarray(np.tile(wnp, _NH)[None, :])
    hf = jnp.asarray(np.tile(hnp, _NH)[None, :])
    wi = jnp.asarray(np.tile(wnp, _NH)[None, :].astype(np.int32))
    lb2 = jnp.asarray((hh * _SEG + lvl_off[ll]).astype(np.int32)[None, :])
    bmul = jnp.asarray(np.full(_K, _NH * _SEG).astype(np.int32)[None, :])

    query2 = query.reshape(_ROWS, _EMBED)
    rp2 = ref_pts.reshape(_ROWS, 4)
    idx, wts = _prep(query2, rp2, wx, wy, W_attn, bx, by, ba, seg, wf, hf,
                     lb2, bmul, wi)

    tbl = jnp.concatenate(
        [v.astype(jnp.bfloat16).transpose(0, 2, 3, 1).reshape(
            _BS * _NH, -1, _HD)
         for v in (value_0, value_1, value_2)], axis=1).reshape(_TBL, _HD)

    idx_flat = idx.reshape(-1)
    wts_flat = wts.reshape(-1)
    out = _sc_gather(tbl, idx_flat, wts_flat)
    return out.reshape(_BS, _LQ, _NH * _HD)


# speedup vs baseline: 1.2847x; 1.0032x over previous
import dataclasses
import functools

import jax
import jax.numpy as jnp
import numpy as np
from jax import lax
from jax.experimental import pallas as pl
from jax.experimental.pallas import tpu as pltpu
from jax.experimental.pallas import tpu_sc as plsc

_EMBED = 256
_NH = 8
_HD = 32
_PTS = 12
_LVLS = ((80, 80), (40, 40), (20, 20))
_BS = 16
_LQ = 1000
_K = _NH * _PTS
_ROWS = _BS * _LQ
_SEG = sum(h * w for h, w in _LVLS)
_TBL = _BS * _NH * _SEG
_G = 4 * _K

_NW = 32
_QPW = _ROWS // _NW
_QB = 5
_NB = _QPW // _QB
_GB = _QB * _G


def _prep_body(q_ref, rp_ref, wx_ref, wy_ref, wa_ref, bx_ref, by_ref, ba_ref,
               seg_ref, wf_ref, hf_ref, lb2_ref, bmul_ref, wi_ref,
               idx_ref, wts_ref):
    b = pl.program_id(0)
    q = q_ref[...]
    offx = jnp.dot(q, wx_ref[...], preferred_element_type=jnp.float32) + bx_ref[...]
    offy = jnp.dot(q, wy_ref[...], preferred_element_type=jnp.float32) + by_ref[...]
    logit = jnp.dot(q, wa_ref[...], preferred_element_type=jnp.float32) + ba_ref[...]

    m = jnp.max(logit, axis=1, keepdims=True)
    e = jnp.exp(logit - m)
    ssum = jnp.dot(e, seg_ref[...], preferred_element_type=jnp.float32)
    attn = e / ssum

    cx = rp_ref[:, 0:1]
    cy = rp_ref[:, 1:2]
    rw = rp_ref[:, 2:3]
    rh = rp_ref[:, 3:4]
    wf = wf_ref[...]
    hf = hf_ref[...]
    x = (cx + offx * (0.125 * rw)) * wf - 0.5
    y = (cy + offy * (0.125 * rh)) * hf - 0.5
    x0 = jnp.floor(x)
    y0 = jnp.floor(y)
    fx1 = x - x0
    fx0 = 1.0 - fx1
    fy1 = y - y0
    fy0 = 1.0 - fy1

    base = b * bmul_ref[...] + lb2_ref[...]
    wi = wi_ref[...]

    idx_parts = []
    wts_parts = []
    for dx, dy, fx, fy in ((0, 0, fx0, fy0), (1, 0, fx1, fy0),
                           (0, 1, fx0, fy1), (1, 1, fx1, fy1)):
        xi = x0 + dx
        yi = y0 + dy
        valid = ((xi >= 0.0) & (xi <= wf - 1.0)
                 & (yi >= 0.0) & (yi <= hf - 1.0))
        xc = jnp.clip(xi, 0.0, wf - 1.0).astype(jnp.int32)
        yc = jnp.clip(yi, 0.0, hf - 1.0).astype(jnp.int32)
        ind = base + yc * wi + xc
        idx_parts.append(jnp.clip(ind, 0, _TBL - 1))
        wts_parts.append(jnp.where(valid, fx * fy, 0.0) * attn)
    idx_ref[...] = jnp.concatenate(idx_parts, axis=1)
    wts_ref[...] = jnp.concatenate(wts_parts, axis=1)


def _prep(query2, rp2, wx, wy, wa, bx, by, ba, seg, wf, hf, lb2, bmul, wi):
    rep = lambda b: (0, 0)
    return pl.pallas_call(
        _prep_body,
        grid=(_BS,),
        in_specs=[
            pl.BlockSpec((_LQ, _EMBED), lambda b: (b, 0)),
            pl.BlockSpec((_LQ, 4), lambda b: (b, 0)),
            pl.BlockSpec((_EMBED, _K), rep),
            pl.BlockSpec((_EMBED, _K), rep),
            pl.BlockSpec((_EMBED, _K), rep),
            pl.BlockSpec((1, _K), rep),
            pl.BlockSpec((1, _K), rep),
            pl.BlockSpec((1, _K), rep),
            pl.BlockSpec((_K, _K), rep),
            pl.BlockSpec((1, _K), rep),
            pl.BlockSpec((1, _K), rep),
            pl.BlockSpec((1, _K), rep),
            pl.BlockSpec((1, _K), rep),
            pl.BlockSpec((1, _K), rep),
        ],
        out_specs=[
            pl.BlockSpec((_LQ, _G), lambda b: (b, 0)),
            pl.BlockSpec((_LQ, _G), lambda b: (b, 0)),
        ],
        out_shape=[
            jax.ShapeDtypeStruct((_ROWS, _G), jnp.int32),
            jax.ShapeDtypeStruct((_ROWS, _G), jnp.float32),
        ],
    )(query2, rp2, wx, wy, wa, bx, by, ba, seg, wf, hf, lb2, bmul, wi)


def _sc_body(tbl_hbm, idx_hbm, wts_hbm, out_hbm,
             idx_v0, idx_v1, idx_v2, wts_v0, wts_v1, wts_v2,
             rows_v0, rows_v1, rows_v2, out_v0, out_v1, out_v2,
             sl0, sl1, sl2, sg0, sg1, sg2, ss0, ss1, ss2):
    c = lax.axis_index("c")
    s = lax.axis_index("s")
    wid = c * 16 + s
    row0 = wid * _QPW
    bufs = ((idx_v0, wts_v0, rows_v0, out_v0, sl0, sg0, ss0),
            (idx_v1, wts_v1, rows_v1, out_v1, sl1, sg1, ss1),
            (idx_v2, wts_v2, rows_v2, out_v2, sl2, sg2, ss2))

    def load(blk, buf):
        idx_v, wts_v, _, _, sl, _, _ = buf
        off = (row0 + blk * _QB) * _G
        pltpu.async_copy(idx_hbm.at[pl.ds(off, _GB)], idx_v, sl)
        pltpu.async_copy(wts_hbm.at[pl.ds(off, _GB)], wts_v, sl)

    def wait_load(buf):
        idx_v, wts_v, _, _, sl, _, _ = buf
        pltpu.make_async_copy(idx_hbm.at[pl.ds(0, _GB)], idx_v, sl).wait()
        pltpu.make_async_copy(wts_hbm.at[pl.ds(0, _GB)], wts_v, sl).wait()

    def gather(buf):
        idx_v, _, rows_v, _, _, sg, _ = buf
        pltpu.async_copy(tbl_hbm.at[idx_v], rows_v, sg)

    def wait_gather(buf):
        idx_v, _, rows_v, _, _, sg, _ = buf
        pltpu.make_async_copy(tbl_hbm.at[idx_v], rows_v, sg).wait()

    def store(blk, buf):
        _, _, _, out_v, _, _, ss = buf
        qrow = row0 + blk * _QB
        pltpu.async_copy(out_v, out_hbm.at[pl.ds(qrow * _NH, _QB * _NH)], ss)

    def wait_store(buf):
        _, _, _, out_v, _, _, ss = buf
        pltpu.make_async_copy(
            out_v, out_hbm.at[pl.ds(0, _QB * _NH)], ss).wait()

    def compute(blk, buf):
        _, wts_v, rows_v, out_v, _, _, _ = buf
        lanes = lax.iota(jnp.int32, 16)
        for qi in range(_QB):
            @pl.loop(0, _NH)
            def _(h):
                jbase = qi * _G + h * _PTS
                a0 = jnp.zeros((16,), jnp.float32)
                a1 = jnp.zeros((16,), jnp.float32)
                for cc in range(4):
                    for p in range(_PTS):
                        j = jbase + cc * _K + p
                        wv = plsc.load_gather(
                            wts_v, [jnp.broadcast_to(j, (16,))])
                        ra, rb = plsc.unpack(rows_v[j],
                                             format=plsc.PackFormat.INTERLEAVED)
                        a0 = a0 + wv * ra
                        a1 = a1 + wv * rb
                orow = out_v.at[qi * _NH + h]
                plsc.store_scatter(orow, [lanes * 2], a0)
                plsc.store_scatter(orow, [lanes * 2 + 1], a1)
        store(blk, buf)

    load(0, bufs[0])
    load(1, bufs[1])
    wait_load(bufs[0])
    gather(bufs[0])

    ngrp = (_NB - 1) // 3

    @pl.loop(0, ngrp)
    def _(i):
        for r in range(3):
            b = 3 * i + r
            buf = bufs[r]
            wait_gather(buf)
            nbuf = bufs[(r + 1) % 3]
            wait_load(nbuf)
            gather(nbuf)

            @pl.when(b + 2 < _NB)
            def _():
                load(b + 2, bufs[(r + 2) % 3])

            @pl.when(b >= 3)
            def _():
                wait_store(buf)

            compute(b, buf)
            store(b, buf)

    wait_gather(bufs[0])
    wait_store(bufs[0])
    compute(_NB - 1, bufs[0])
    store(_NB - 1, bufs[0])

    wait_store(bufs[0])
    wait_store(bufs[1])
    wait_store(bufs[2])


def _sc_gather(tbl, idx3, wts_flat):
    mesh = plsc.VectorSubcoreMesh(core_axis_name="c", subcore_axis_name="s")
    cp = pltpu.CompilerParams()
    if "needs_layout_passes" in pltpu.CompilerParams.__dataclass_fields__:
        cp = dataclasses.replace(cp, needs_layout_passes=False)
    if "use_tc_tiling_on_sc" in pltpu.CompilerParams.__dataclass_fields__:
        cp = dataclasses.replace(cp, use_tc_tiling_on_sc=False)
    f = pl.kernel(
        _sc_body,
        mesh=mesh,
        compiler_params=cp,
        out_type=jax.ShapeDtypeStruct((_ROWS * _NH, _HD), jnp.float32),
        scratch_types=(
            [pltpu.VMEM((_GB,), jnp.int32)] * 3
            + [pltpu.VMEM((_GB,), jnp.float32)] * 3
            + [pltpu.VMEM((_GB, _HD), jnp.bfloat16)] * 3
            + [pltpu.VMEM((_QB * _NH, _HD), jnp.float32)] * 3
            + [pltpu.SemaphoreType.DMA] * 9
        ),
    )
    return f(tbl, idx3, wts_flat)


def kernel(query, ref_pts, value_0, value_1, value_2, W_off, b_off, W_attn,
           b_attn):
    wx = W_off[:, 0::2]
    wy = W_off[:, 1::2]
    bx = b_off[0::2][None, :]
    by = b_off[1::2][None, :]
    ba = b_attn[None, :]

    seg = jnp.asarray(np.kron(np.eye(_NH, dtype=np.float32),
                              np.ones((_PTS, _PTS), np.float32)))
    lvl_of_p = np.repeat(np.arange(3), 4)
    hw_np = np.array([h * w for h, w in _LVLS], np.int64)
    wnp = np.array([w for _, w in _LVLS], np.float32)[lvl_of_p]
    hnp = np.array([h for h, _ in _LVLS], np.float32)[lvl_of_p]
    lvl_off = np.concatenate(([0], np.cumsum(hw_np)[:-1]))
    kk = np.arange(_K)
    hh = kk // _PTS
    ll = lvl_of_p[kk % _PTS]
    wf = jnp.asarray(np.tile(wnp, _NH)[None, :])
    hf = jnp.asarray(np.tile(hnp, _NH)[None, :])
    wi = jnp.asarray(np.tile(wnp, _NH)[None, :].astype(np.int32))
    lb2 = jnp.asarray((hh * _SEG + lvl_off[ll]).astype(np.int32)[None, :])
    bmul = jnp.asarray(np.full(_K, _NH * _SEG).astype(np.int32)[None, :])

    query2 = query.reshape(_ROWS, _EMBED)
    rp2 = ref_pts.reshape(_ROWS, 4)
    idx, wts = _prep(query2, rp2, wx, wy, W_attn, bx, by, ba, seg, wf, hf,
                     lb2, bmul, wi)

    tbl = jnp.concatenate(
        [v.astype(jnp.bfloat16).transpose(0, 2, 3, 1).reshape(
            _BS * _NH, -1, _HD)
         for v in (value_0, value_1, value_2)], axis=1).reshape(_TBL, _HD)

    idx_flat = idx.reshape(-1)
    wts_flat = wts.reshape(-1)
    out = _sc_gather(tbl, idx_flat, wts_flat)
    return out.reshape(_BS, _LQ, _NH * _HD)


# speedup vs baseline: 1.3530x; 1.0532x over previous
import dataclasses
import functools

import jax
import jax.numpy as jnp
import numpy as np
from jax import lax
from jax.experimental import pallas as pl
from jax.experimental.pallas import tpu as pltpu
from jax.experimental.pallas import tpu_sc as plsc

_EMBED = 256
_NH = 8
_HD = 32
_PTS = 12
_LVLS = ((80, 80), (40, 40), (20, 20))
_BS = 16
_LQ = 1000
_K = _NH * _PTS
_ROWS = _BS * _LQ
_SEG = sum(h * w for h, w in _LVLS)
_TBL = _BS * _NH * _SEG
_G = 4 * _K

_NW = 32
_QPW = _ROWS // _NW
_QB = 5
_NB = _QPW // _QB
_GB = _QB * _G


def _prep_body(q_ref, rp_ref, wx_ref, wy_ref, wa_ref, bx_ref, by_ref, ba_ref,
               seg_ref, wf_ref, hf_ref, lb2_ref, bmul_ref, wi_ref,
               idx_ref, wts_ref):
    b = pl.program_id(0)
    q = q_ref[...]
    offx = jnp.dot(q, wx_ref[...], preferred_element_type=jnp.float32) + bx_ref[...]
    offy = jnp.dot(q, wy_ref[...], preferred_element_type=jnp.float32) + by_ref[...]
    logit = jnp.dot(q, wa_ref[...], preferred_element_type=jnp.float32) + ba_ref[...]

    m = jnp.max(logit, axis=1, keepdims=True)
    e = jnp.exp(logit - m)
    ssum = jnp.dot(e, seg_ref[...], preferred_element_type=jnp.float32)
    attn = e / ssum

    cx = rp_ref[:, 0:1]
    cy = rp_ref[:, 1:2]
    rw = rp_ref[:, 2:3]
    rh = rp_ref[:, 3:4]
    wf = wf_ref[...]
    hf = hf_ref[...]
    x = (cx + offx * (0.125 * rw)) * wf - 0.5
    y = (cy + offy * (0.125 * rh)) * hf - 0.5
    x0 = jnp.floor(x)
    y0 = jnp.floor(y)
    fx1 = x - x0
    fx0 = 1.0 - fx1
    fy1 = y - y0
    fy0 = 1.0 - fy1

    base = b * bmul_ref[...] + lb2_ref[...]
    wi = wi_ref[...]

    idx_parts = []
    wts_parts = []
    for dx, dy, fx, fy in ((0, 0, fx0, fy0), (1, 0, fx1, fy0),
                           (0, 1, fx0, fy1), (1, 1, fx1, fy1)):
        xi = x0 + dx
        yi = y0 + dy
        valid = ((xi >= 0.0) & (xi <= wf - 1.0)
                 & (yi >= 0.0) & (yi <= hf - 1.0))
        xc = jnp.clip(xi, 0.0, wf - 1.0).astype(jnp.int32)
        yc = jnp.clip(yi, 0.0, hf - 1.0).astype(jnp.int32)
        ind = base + yc * wi + xc
        idx_parts.append(jnp.clip(ind, 0, _TBL - 1))
        wts_parts.append(jnp.where(valid, fx * fy, 0.0) * attn)
    idx_ref[...] = jnp.concatenate(idx_parts, axis=1)
    wts_ref[...] = jnp.concatenate(wts_parts, axis=1)


def _prep(query2, rp2, wx, wy, wa, bx, by, ba, seg, wf, hf, lb2, bmul, wi):
    rep = lambda b: (0, 0)
    return pl.pallas_call(
        _prep_body,
        grid=(_BS,),
        in_specs=[
            pl.BlockSpec((_LQ, _EMBED), lambda b: (b, 0)),
            pl.BlockSpec((_LQ, 4), lambda b: (b, 0)),
            pl.BlockSpec((_EMBED, _K), rep),
            pl.BlockSpec((_EMBED, _K), rep),
            pl.BlockSpec((_EMBED, _K), rep),
            pl.BlockSpec((1, _K), rep),
            pl.BlockSpec((1, _K), rep),
            pl.BlockSpec((1, _K), rep),
            pl.BlockSpec((_K, _K), rep),
            pl.BlockSpec((1, _K), rep),
            pl.BlockSpec((1, _K), rep),
            pl.BlockSpec((1, _K), rep),
            pl.BlockSpec((1, _K), rep),
            pl.BlockSpec((1, _K), rep),
        ],
        out_specs=[
            pl.BlockSpec((_LQ, _G), lambda b: (b, 0)),
            pl.BlockSpec((_LQ, _G), lambda b: (b, 0)),
        ],
        out_shape=[
            jax.ShapeDtypeStruct((_ROWS, _G), jnp.int32),
            jax.ShapeDtypeStruct((_ROWS, _G), jnp.float32),
        ],
    )(query2, rp2, wx, wy, wa, bx, by, ba, seg, wf, hf, lb2, bmul, wi)


def _sc_body(tbl_hbm, idx_hbm, wts_hbm, out_hbm,
             idx_v0, idx_v1, idx_v2, wts_v0, wts_v1, wts_v2,
             rows_v0, rows_v1, rows_v2, out_v0, out_v1, out_v2,
             sl0, sl1, sl2, sg0, sg1, sg2, ss0, ss1, ss2):
    c = lax.axis_index("c")
    s = lax.axis_index("s")
    wid = c * 16 + s
    row0 = wid * _QPW
    bufs = ((idx_v0, wts_v0, rows_v0, out_v0, sl0, sg0, ss0),
            (idx_v1, wts_v1, rows_v1, out_v1, sl1, sg1, ss1),
            (idx_v2, wts_v2, rows_v2, out_v2, sl2, sg2, ss2))

    def load(blk, buf):
        idx_v, wts_v, _, _, sl, _, _ = buf
        off = (row0 + blk * _QB) * _G
        pltpu.async_copy(idx_hbm.at[pl.ds(off, _GB)], idx_v, sl)
        pltpu.async_copy(wts_hbm.at[pl.ds(off, _GB)],
                         wts_v.at[pl.ds(0, _GB)], sl)

    def wait_load(buf):
        idx_v, wts_v, _, _, sl, _, _ = buf
        pltpu.make_async_copy(idx_hbm.at[pl.ds(0, _GB)], idx_v, sl).wait()
        pltpu.make_async_copy(wts_hbm.at[pl.ds(0, _GB)],
                              wts_v.at[pl.ds(0, _GB)], sl).wait()

    def gather(buf):
        idx_v, _, rows_v, _, _, sg, _ = buf
        pltpu.async_copy(tbl_hbm.at[idx_v], rows_v, sg)

    def wait_gather(buf):
        idx_v, _, rows_v, _, _, sg, _ = buf
        pltpu.make_async_copy(tbl_hbm.at[idx_v], rows_v, sg).wait()

    def store(blk, buf):
        _, _, _, out_v, _, _, ss = buf
        qrow = row0 + blk * _QB
        pltpu.async_copy(out_v, out_hbm.at[pl.ds(qrow * _NH, _QB * _NH)], ss)

    def wait_store(buf):
        _, _, _, out_v, _, _, ss = buf
        pltpu.make_async_copy(
            out_v, out_hbm.at[pl.ds(0, _QB * _NH)], ss).wait()

    def compute(blk, buf):
        _, wts_v, rows_v, out_v, _, _, _ = buf
        lanes = lax.iota(jnp.int32, 16)
        for qi in range(_QB):
            @pl.loop(0, _NH)
            def _(h):
                jbase = qi * _G + h * _PTS
                a0 = jnp.zeros((16,), jnp.float32)
                a1 = jnp.zeros((16,), jnp.float32)
                for cc in range(4):
                    jc = jbase + cc * _K
                    wv16 = wts_v[pl.ds(jc, 16)]
                    for p in range(_PTS):
                        wv = jnp.broadcast_to(wv16[p], (16,))
                        ra, rb = plsc.unpack(rows_v[jc + p],
                                             format=plsc.PackFormat.INTERLEAVED)
                        a0 = a0 + wv * ra
                        a1 = a1 + wv * rb
                orow = out_v.at[qi * _NH + h]
                plsc.store_scatter(orow, [lanes * 2], a0)
                plsc.store_scatter(orow, [lanes * 2 + 1], a1)
        store(blk, buf)

    load(0, bufs[0])
    load(1, bufs[1])
    wait_load(bufs[0])
    gather(bufs[0])

    ngrp = (_NB - 1) // 3

    @pl.loop(0, ngrp)
    def _(i):
        for r in range(3):
            b = 3 * i + r
            buf = bufs[r]
            wait_gather(buf)
            nbuf = bufs[(r + 1) % 3]
            wait_load(nbuf)
            gather(nbuf)

            @pl.when(b + 2 < _NB)
            def _():
                load(b + 2, bufs[(r + 2) % 3])

            @pl.when(b >= 3)
            def _():
                wait_store(buf)

            compute(b, buf)
            store(b, buf)

    wait_gather(bufs[0])
    wait_store(bufs[0])
    compute(_NB - 1, bufs[0])
    store(_NB - 1, bufs[0])

    wait_store(bufs[0])
    wait_store(bufs[1])
    wait_store(bufs[2])


def _sc_gather(tbl, idx3, wts_flat):
    mesh = plsc.VectorSubcoreMesh(core_axis_name="c", subcore_axis_name="s")
    cp = pltpu.CompilerParams()
    if "needs_layout_passes" in pltpu.CompilerParams.__dataclass_fields__:
        cp = dataclasses.replace(cp, needs_layout_passes=False)
    if "use_tc_tiling_on_sc" in pltpu.CompilerParams.__dataclass_fields__:
        cp = dataclasses.replace(cp, use_tc_tiling_on_sc=False)
    f = pl.kernel(
        _sc_body,
        mesh=mesh,
        compiler_params=cp,
        out_type=jax.ShapeDtypeStruct((_ROWS * _NH, _HD), jnp.float32),
        scratch_types=(
            [pltpu.VMEM((_GB,), jnp.int32)] * 3
            + [pltpu.VMEM((_GB + 16,), jnp.float32)] * 3
            + [pltpu.VMEM((_GB, _HD), jnp.bfloat16)] * 3
            + [pltpu.VMEM((_QB * _NH, _HD), jnp.float32)] * 3
            + [pltpu.SemaphoreType.DMA] * 9
        ),
    )
    return f(tbl, idx3, wts_flat)


def kernel(query, ref_pts, value_0, value_1, value_2, W_off, b_off, W_attn,
           b_attn):
    wx = W_off[:, 0::2]
    wy = W_off[:, 1::2]
    bx = b_off[0::2][None, :]
    by = b_off[1::2][None, :]
    ba = b_attn[None, :]

    seg = jnp.asarray(np.kron(np.eye(_NH, dtype=np.float32),
                              np.ones((_PTS, _PTS), np.float32)))
    lvl_of_p = np.repeat(np.arange(3), 4)
    hw_np = np.array([h * w for h, w in _LVLS], np.int64)
    wnp = np.array([w for _, w in _LVLS], np.float32)[lvl_of_p]
    hnp = np.array([h for h, _ in _LVLS], np.float32)[lvl_of_p]
    lvl_off = np.concatenate(([0], np.cumsum(hw_np)[:-1]))
    kk = np.arange(_K)
    hh = kk // _PTS
    ll = lvl_of_p[kk % _PTS]
    wf = jnp.asarray(np.tile(wnp, _NH)[None, :])
    hf = jnp.asarray(np.tile(hnp, _NH)[None, :])
    wi = jnp.asarray(np.tile(wnp, _NH)[None, :].astype(np.int32))
    lb2 = jnp.asarray((hh * _SEG + lvl_off[ll]).astype(np.int32)[None, :])
    bmul = jnp.asarray(np.full(_K, _NH * _SEG).astype(np.int32)[None, :])

    query2 = query.reshape(_ROWS, _EMBED)
    rp2 = ref_pts.reshape(_ROWS, 4)
    idx, wts = _prep(query2, rp2, wx, wy, W_attn, bx, by, ba, seg, wf, hf,
                     lb2, bmul, wi)

    tbl = jnp.concatenate(
        [v.astype(jnp.bfloat16).transpose(0, 2, 3, 1).reshape(
            _BS * _NH, -1, _HD)
         for v in (value_0, value_1, value_2)], axis=1).reshape(_TBL, _HD)

    idx_flat = idx.reshape(-1)
    wts_flat = wts.reshape(-1)
    out = _sc_gather(tbl, idx_flat, wts_flat)
    return out.reshape(_BS, _LQ, _NH * _HD)


# speedup vs baseline: 1.3537x; 1.0005x over previous
import dataclasses

import jax
import jax.numpy as jnp
import numpy as np
from jax import lax
from jax.experimental import pallas as pl
from jax.experimental.pallas import tpu as pltpu
from jax.experimental.pallas import tpu_sc as plsc

_EMBED = 256
_NH = 8
_HD = 32
_PTS = 12
_LVLS = ((80, 80), (40, 40), (20, 20))
_BS = 16
_LQ = 1000
_K = _NH * _PTS
_ROWS = _BS * _LQ
_SEG = sum(h * w for h, w in _LVLS)
_TBL = _BS * _NH * _SEG
_G = 4 * _K

_NW = 32
_QPW = _ROWS // _NW
_QB = 5
_NB = _QPW // _QB
_GB = _QB * _G


def _prep_body(q_ref, rp_ref, wx_ref, wy_ref, wa_ref, bx_ref, by_ref, ba_ref,
               seg_ref, wf_ref, hf_ref, lb2_ref, bmul_ref, wi_ref,
               idx_ref, wts_ref):
    b = pl.program_id(0)
    q = q_ref[...]
    offx = jnp.dot(q, wx_ref[...], preferred_element_type=jnp.float32) + bx_ref[...]
    offy = jnp.dot(q, wy_ref[...], preferred_element_type=jnp.float32) + by_ref[...]
    logit = jnp.dot(q, wa_ref[...], preferred_element_type=jnp.float32) + ba_ref[...]

    m = jnp.max(logit, axis=1, keepdims=True)
    e = jnp.exp(logit - m)
    ssum = jnp.dot(e, seg_ref[...], preferred_element_type=jnp.float32)
    attn = e / ssum

    cx = rp_ref[:, 0:1]
    cy = rp_ref[:, 1:2]
    rw = rp_ref[:, 2:3]
    rh = rp_ref[:, 3:4]
    wf = wf_ref[...]
    hf = hf_ref[...]
    x = (cx + offx * (0.125 * rw)) * wf - 0.5
    y = (cy + offy * (0.125 * rh)) * hf - 0.5
    x0 = jnp.floor(x)
    y0 = jnp.floor(y)
    fx1 = x - x0
    fx0 = 1.0 - fx1
    fy1 = y - y0
    fy0 = 1.0 - fy1

    base = b * bmul_ref[...] + lb2_ref[...]
    wi = wi_ref[...]

    idx_parts = []
    wts_parts = []
    for dx, dy, fx, fy in ((0, 0, fx0, fy0), (1, 0, fx1, fy0),
                           (0, 1, fx0, fy1), (1, 1, fx1, fy1)):
        xi = x0 + dx
        yi = y0 + dy
        valid = ((xi >= 0.0) & (xi <= wf - 1.0)
                 & (yi >= 0.0) & (yi <= hf - 1.0))
        xc = jnp.clip(xi, 0.0, wf - 1.0).astype(jnp.int32)
        yc = jnp.clip(yi, 0.0, hf - 1.0).astype(jnp.int32)
        ind = base + yc * wi + xc
        idx_parts.append(jnp.clip(ind, 0, _TBL - 1))
        wts_parts.append(jnp.where(valid, fx * fy, 0.0) * attn)
    idx_ref[...] = jnp.concatenate(idx_parts, axis=1)
    wts_ref[...] = jnp.concatenate(wts_parts, axis=1)


def _prep(query2, rp2, wx, wy, wa, bx, by, ba, seg, wf, hf, lb2, bmul, wi):
    rep = lambda b: (0, 0)
    return pl.pallas_call(
        _prep_body,
        grid=(_BS,),
        in_specs=[
            pl.BlockSpec((_LQ, _EMBED), lambda b: (b, 0)),
            pl.BlockSpec((_LQ, 4), lambda b: (b, 0)),
            pl.BlockSpec((_EMBED, _K), rep),
            pl.BlockSpec((_EMBED, _K), rep),
            pl.BlockSpec((_EMBED, _K), rep),
            pl.BlockSpec((1, _K), rep),
            pl.BlockSpec((1, _K), rep),
            pl.BlockSpec((1, _K), rep),
            pl.BlockSpec((_K, _K), rep),
            pl.BlockSpec((1, _K), rep),
            pl.BlockSpec((1, _K), rep),
            pl.BlockSpec((1, _K), rep),
            pl.BlockSpec((1, _K), rep),
            pl.BlockSpec((1, _K), rep),
        ],
        out_specs=[
            pl.BlockSpec((_LQ, _G), lambda b: (b, 0)),
            pl.BlockSpec((_LQ, _G), lambda b: (b, 0)),
        ],
        out_shape=[
            jax.ShapeDtypeStruct((_ROWS, _G), jnp.int32),
            jax.ShapeDtypeStruct((_ROWS, _G), jnp.float32),
        ],
    )(query2, rp2, wx, wy, wa, bx, by, ba, seg, wf, hf, lb2, bmul, wi)


def _sc_body(tbl_hbm, idx_hbm, wts_hbm, out_hbm,
             idx_v0, idx_v1, idx_v2, wts_v0, wts_v1, wts_v2,
             rows_v0, rows_v1, rows_v2, out_v0, out_v1, out_v2,
             sl0, sl1, sl2, sg0, sg1, sg2, ss0, ss1, ss2):
    c = lax.axis_index("c")
    s = lax.axis_index("s")
    wid = c * 16 + s
    row0 = wid * _QPW
    bufs = ((idx_v0, wts_v0, rows_v0, out_v0, sl0, sg0, ss0),
            (idx_v1, wts_v1, rows_v1, out_v1, sl1, sg1, ss1),
            (idx_v2, wts_v2, rows_v2, out_v2, sl2, sg2, ss2))

    def load(blk, buf):
        idx_v, wts_v, _, _, sl, _, _ = buf
        off = (row0 + blk * _QB) * _G
        pltpu.async_copy(idx_hbm.at[pl.ds(off, _GB)], idx_v, sl)
        pltpu.async_copy(wts_hbm.at[pl.ds(off, _GB)],
                         wts_v.at[pl.ds(0, _GB)], sl)

    def wait_load(buf):
        idx_v, wts_v, _, _, sl, _, _ = buf
        pltpu.make_async_copy(idx_hbm.at[pl.ds(0, _GB)], idx_v, sl).wait()
        pltpu.make_async_copy(wts_hbm.at[pl.ds(0, _GB)],
                              wts_v.at[pl.ds(0, _GB)], sl).wait()

    def gather(buf):
        idx_v, _, rows_v, _, _, sg, _ = buf
        pltpu.async_copy(tbl_hbm.at[idx_v], rows_v, sg)

    def wait_gather(buf):
        idx_v, _, rows_v, _, _, sg, _ = buf
        pltpu.make_async_copy(tbl_hbm.at[idx_v], rows_v, sg).wait()

    def store(blk, buf):
        _, _, _, out_v, _, _, ss = buf
        qrow = row0 + blk * _QB
        pltpu.async_copy(out_v, out_hbm.at[pl.ds(qrow * _NH, _QB * _NH)], ss)

    def wait_store(buf):
        _, _, _, out_v, _, _, ss = buf
        pltpu.make_async_copy(
            out_v, out_hbm.at[pl.ds(0, _QB * _NH)], ss).wait()

    def compute(blk, buf):
        _, wts_v, rows_v, out_v, _, _, _ = buf
        lanes = lax.iota(jnp.int32, 16)
        for qi in range(_QB):
            @pl.loop(0, _NH)
            def _(h):
                jbase = qi * _G + h * _PTS
                a0 = jnp.zeros((16,), jnp.float32)
                a1 = jnp.zeros((16,), jnp.float32)
                for cc in range(4):
                    jc = jbase + cc * _K
                    wv16 = wts_v[pl.ds(jc, 16)]
                    for p in range(_PTS):
                        wv = jnp.broadcast_to(wv16[p], (16,))
                        ra, rb = plsc.unpack(rows_v[jc + p],
                                             format=plsc.PackFormat.INTERLEAVED)
                        a0 = a0 + wv * ra
                        a1 = a1 + wv * rb
                orow = out_v.at[qi * _NH + h]
                plsc.store_scatter(orow, [lanes * 2], a0)
                plsc.store_scatter(orow, [lanes * 2 + 1], a1)
        store(blk, buf)

    load(0, bufs[0])
    load(1, bufs[1])
    wait_load(bufs[0])
    gather(bufs[0])

    ngrp = (_NB - 1) // 3

    @pl.loop(0, ngrp)
    def _(i):
        for r in range(3):
            b = 3 * i + r
            buf = bufs[r]
            wait_gather(buf)
            nbuf = bufs[(r + 1) % 3]
            wait_load(nbuf)
            gather(nbuf)

            @pl.when(b + 2 < _NB)
            def _():
                load(b + 2, bufs[(r + 2) % 3])

            @pl.when(b >= 3)
            def _():
                wait_store(buf)

            compute(b, buf)
            store(b, buf)

    wait_gather(bufs[0])
    wait_store(bufs[0])
    compute(_NB - 1, bufs[0])
    store(_NB - 1, bufs[0])

    wait_store(bufs[0])
    wait_store(bufs[1])
    wait_store(bufs[2])


def _sc_gather(tbl, idx3, wts_flat):
    mesh = plsc.VectorSubcoreMesh(core_axis_name="c", subcore_axis_name="s")
    cp = pltpu.CompilerParams()
    if "needs_layout_passes" in pltpu.CompilerParams.__dataclass_fields__:
        cp = dataclasses.replace(cp, needs_layout_passes=False)
    if "use_tc_tiling_on_sc" in pltpu.CompilerParams.__dataclass_fields__:
        cp = dataclasses.replace(cp, use_tc_tiling_on_sc=False)
    f = pl.kernel(
        _sc_body,
        mesh=mesh,
        compiler_params=cp,
        out_type=jax.ShapeDtypeStruct((_ROWS * _NH, _HD), jnp.float32),
        scratch_types=(
            [pltpu.VMEM((_GB,), jnp.int32)] * 3
            + [pltpu.VMEM((_GB + 16,), jnp.float32)] * 3
            + [pltpu.VMEM((_GB, _HD), jnp.bfloat16)] * 3
            + [pltpu.VMEM((_QB * _NH, _HD), jnp.float32)] * 3
            + [pltpu.SemaphoreType.DMA] * 9
        ),
    )
    return f(tbl, idx3, wts_flat)


def kernel(query, ref_pts, value_0, value_1, value_2, W_off, b_off, W_attn,
           b_attn):
    wx = W_off[:, 0::2]
    wy = W_off[:, 1::2]
    bx = b_off[0::2][None, :]
    by = b_off[1::2][None, :]
    ba = b_attn[None, :]

    seg = jnp.asarray(np.kron(np.eye(_NH, dtype=np.float32),
                              np.ones((_PTS, _PTS), np.float32)))
    lvl_of_p = np.repeat(np.arange(3), 4)
    hw_np = np.array([h * w for h, w in _LVLS], np.int64)
    wnp = np.array([w for _, w in _LVLS], np.float32)[lvl_of_p]
    hnp = np.array([h for h, _ in _LVLS], np.float32)[lvl_of_p]
    lvl_off = np.concatenate(([0], np.cumsum(hw_np)[:-1]))
    kk = np.arange(_K)
    hh = kk // _PTS
    ll = lvl_of_p[kk % _PTS]
    wf = jnp.asarray(np.tile(wnp, _NH)[None, :])
    hf = jnp.asarray(np.tile(hnp, _NH)[None, :])
    wi = jnp.asarray(np.tile(wnp, _NH)[None, :].astype(np.int32))
    lb2 = jnp.asarray((hh * _SEG + lvl_off[ll]).astype(np.int32)[None, :])
    bmul = jnp.asarray(np.full(_K, _NH * _SEG).astype(np.int32)[None, :])

    query2 = query.reshape(_ROWS, _EMBED)
    rp2 = ref_pts.reshape(_ROWS, 4)
    idx, wts = _prep(query2, rp2, wx, wy, W_attn, bx, by, ba, seg, wf, hf,
                     lb2, bmul, wi)

    tbl = jnp.concatenate(
        [v.astype(jnp.bfloat16).transpose(0, 2, 3, 1).reshape(
            _BS * _NH, -1, _HD)
         for v in (value_0, value_1, value_2)], axis=1).reshape(_TBL, _HD)

    idx_flat = idx.reshape(-1)
    wts_flat = wts.reshape(-1)
    out = _sc_gather(tbl, idx_flat, wts_flat)
    return out.reshape(_BS, _LQ, _NH * _HD)
